# Initial kernel scaffold; baseline (speedup 1.0000x reference)
import jax
import jax.numpy as jnp
from jax.experimental import pallas as pl


def kernel(src_table, tgt_table, src_indices, tgt_indices):
    raise NotImplementedError("write your pallas kernel here")


# speedup vs baseline: 2.8806x; 2.8806x over previous
import functools

import jax
import jax.numpy as jnp
from jax import lax
from jax.experimental import pallas as pl
from jax.experimental.pallas import tpu as pltpu
from jax.experimental.pallas import tpu_sc as plsc

_CHUNK = 128


@functools.partial(jax.jit, static_argnames=())
def _sc_lookup(src_table, tgt_table, src_idx, tgt_idx):
    V, D = src_table.shape
    N = src_idx.shape[0]
    info = plsc.get_sparse_core_info()
    NC, NS = info.num_cores, info.num_subcores
    NW = NC * NS
    per_w = N // NW
    n_chunks = per_w // _CHUNK

    mesh = plsc.VectorSubcoreMesh(core_axis_name="c", subcore_axis_name="s")

    @functools.partial(
        pl.kernel,
        mesh=mesh,
        out_type=[
            jax.ShapeDtypeStruct((N, D), jnp.float32),
            jax.ShapeDtypeStruct((N, D), jnp.float32),
        ],
        scratch_types=[
            pltpu.VMEM((_CHUNK,), jnp.int32),
            pltpu.VMEM((_CHUNK, D), jnp.float32),
            pltpu.SemaphoreType.DMA,
        ],
    )
    def k(src_t, tgt_t, src_i, tgt_i, src_o, tgt_o, idx_v, rows_v, sem):
        wid = lax.axis_index("s") * NC + lax.axis_index("c")
        base = wid * per_w

        def side(table, idx_hbm, out_hbm):
            def body(g, carry):
                off = base + g * _CHUNK
                pltpu.sync_copy(idx_hbm.at[pl.ds(off, _CHUNK)], idx_v)
                pltpu.async_copy(table.at[idx_v], rows_v, sem).wait()
                pltpu.sync_copy(rows_v, out_hbm.at[pl.ds(off, _CHUNK)])
                return carry

            lax.fori_loop(0, n_chunks, body, 0)

        side(src_t, src_i, src_o)
        side(tgt_t, tgt_i, tgt_o)

    return k(src_table, tgt_table, src_idx, tgt_idx)


def kernel(src_table, tgt_table, src_indices, tgt_indices):
    B, L = src_indices.shape
    D = src_table.shape[1]
    src_flat = src_indices.reshape(-1).astype(jnp.int32)
    tgt_flat = tgt_indices.reshape(-1).astype(jnp.int32)
    src_out, tgt_out = _sc_lookup(src_table, tgt_table, src_flat, tgt_flat)
    return src_out.reshape(B, L, D), tgt_out.reshape(B, L, D)


# speedup vs baseline: 3.5354x; 1.2273x over previous
import functools

import jax
import jax.numpy as jnp
from jax import lax
from jax.experimental import pallas as pl
from jax.experimental.pallas import tpu as pltpu
from jax.experimental.pallas import tpu_sc as plsc

_CHUNK = 128
_K = 5


def _sc_lookup(src_table, tgt_table, src_idx, tgt_idx):
    V, D = src_table.shape
    NW, NCH, _ = src_idx.shape
    N = NW * NCH * _CHUNK
    per_w = NCH * _CHUNK
    n_outer = NCH // _K

    mesh = plsc.VectorSubcoreMesh(core_axis_name="c", subcore_axis_name="s")
    info = plsc.get_sparse_core_info()
    NC = info.num_cores

    @functools.partial(
        pl.kernel,
        mesh=mesh,
        out_type=[
            jax.ShapeDtypeStruct((N, D), jnp.float32),
            jax.ShapeDtypeStruct((N, D), jnp.float32),
        ],
        scratch_types=[
            pltpu.VMEM((NCH, _CHUNK), jnp.int32),
            pltpu.VMEM((NCH, _CHUNK), jnp.int32),
            pltpu.VMEM((_K, _CHUNK, D), jnp.float32),
            pltpu.SemaphoreType.DMA((_K,)),
            pltpu.SemaphoreType.DMA((_K,)),
        ],
    )
    def k(src_t, tgt_t, src_i, tgt_i, src_o, tgt_o,
          idx_src_v, idx_tgt_v, rows_v, sem_g, sem_w):
        wid = lax.axis_index("s") * NC + lax.axis_index("c")
        base = wid * per_w

        pltpu.sync_copy(src_i.at[wid], idx_src_v)
        pltpu.sync_copy(tgt_i.at[wid], idx_tgt_v)

        def side(table, idx_v, out_hbm, prev_out):
            def outer(t, carry):
                gathers = []
                for b in range(_K):
                    @pl.when(t > 0)
                    def _():
                        pltpu.make_async_copy(
                            rows_v.at[b],
                            out_hbm.at[pl.ds(base, _CHUNK)],
                            sem_w.at[b],
                        ).wait()

                    if prev_out is not None:
                        @pl.when(t == 0)
                        def _():
                            pltpu.make_async_copy(
                                rows_v.at[b],
                                prev_out.at[pl.ds(base, _CHUNK)],
                                sem_w.at[b],
                            ).wait()

                    c = t * _K + b
                    gathers.append(pltpu.async_copy(
                        table.at[idx_v.at[c]], rows_v.at[b], sem_g.at[b]))

                for b in range(_K):
                    gathers[b].wait()
                    c = t * _K + b
                    pltpu.async_copy(
                        rows_v.at[b],
                        out_hbm.at[pl.ds(base + c * _CHUNK, _CHUNK)],
                        sem_w.at[b],
                    )
                return carry

            lax.fori_loop(0, n_outer, outer, 0)

        side(src_t, idx_src_v, src_o, None)
        side(tgt_t, idx_tgt_v, tgt_o, src_o)

        for b in range(_K):
            pltpu.make_async_copy(
                rows_v.at[b], tgt_o.at[pl.ds(base, _CHUNK)], sem_w.at[b]
            ).wait()

    return k(src_table, tgt_table, src_idx, tgt_idx)


def kernel(src_table, tgt_table, src_indices, tgt_indices):
    B, L = src_indices.shape
    D = src_table.shape[1]
    info = plsc.get_sparse_core_info()
    NW = info.num_cores * info.num_subcores
    NCH = (B * L) // (NW * _CHUNK)
    src_flat = src_indices.reshape(NW, NCH, _CHUNK).astype(jnp.int32)
    tgt_flat = tgt_indices.reshape(NW, NCH, _CHUNK).astype(jnp.int32)
    src_out, tgt_out = _sc_lookup(src_table, tgt_table, src_flat, tgt_flat)
    return src_out.reshape(B, L, D), tgt_out.reshape(B, L, D)


# speedup vs baseline: 5.9554x; 1.6845x over previous
import functools

import jax
import jax.numpy as jnp
from jax import lax
from jax.experimental import pallas as pl
from jax.experimental.pallas import tpu as pltpu
from jax.experimental.pallas import tpu_sc as plsc

_K = 8


def _sc_lookup(src_table, tgt_table, src_idx, tgt_idx):
    V, D = src_table.shape
    B, L = src_idx.shape
    info = plsc.get_sparse_core_info()
    NC = info.num_cores
    NW = NC * info.num_subcores
    per_w = B // NW
    n_outer = per_w // _K

    mesh = plsc.VectorSubcoreMesh(core_axis_name="c", subcore_axis_name="s")

    @functools.partial(
        pl.kernel,
        mesh=mesh,
        out_type=[
            jax.ShapeDtypeStruct((B, L, D), jnp.float32),
            jax.ShapeDtypeStruct((B, L, D), jnp.float32),
        ],
        scratch_types=[
            pltpu.VMEM((per_w, L), jnp.int32),
            pltpu.VMEM((per_w, L), jnp.int32),
            pltpu.VMEM((_K, L, D), jnp.float32),
            pltpu.SemaphoreType.DMA((_K,)),
            pltpu.SemaphoreType.DMA((_K,)),
        ],
    )
    def k(src_t, tgt_t, src_i, tgt_i, src_o, tgt_o,
          idx_src_v, idx_tgt_v, rows_v, sem_g, sem_w):
        wid = lax.axis_index("s") * NC + lax.axis_index("c")
        base = wid * per_w

        pltpu.sync_copy(src_i.at[pl.ds(base, per_w)], idx_src_v)
        pltpu.sync_copy(tgt_i.at[pl.ds(base, per_w)], idx_tgt_v)

        def side(table, idx_v, out_hbm, prev_out):
            def outer(t, carry):
                gathers = []
                for b in range(_K):
                    @pl.when(t > 0)
                    def _():
                        pltpu.make_async_copy(
                            rows_v.at[b], out_hbm.at[base], sem_w.at[b]
                        ).wait()

                    if prev_out is not None:
                        @pl.when(t == 0)
                        def _():
                            pltpu.make_async_copy(
                                rows_v.at[b], prev_out.at[base], sem_w.at[b]
                            ).wait()

                    c = t * _K + b
                    gathers.append(pltpu.async_copy(
                        table.at[idx_v.at[c]], rows_v.at[b], sem_g.at[b]))

                for b in range(_K):
                    gathers[b].wait()
                    c = t * _K + b
                    pltpu.async_copy(
                        rows_v.at[b], out_hbm.at[base + c], sem_w.at[b])
                return carry

            lax.fori_loop(0, n_outer, outer, 0)

        side(src_t, idx_src_v, src_o, None)
        side(tgt_t, idx_tgt_v, tgt_o, src_o)

        for b in range(_K):
            pltpu.make_async_copy(
                rows_v.at[b], tgt_o.at[base], sem_w.at[b]
            ).wait()

    return k(src_table, tgt_table, src_idx, tgt_idx)


def kernel(src_table, tgt_table, src_indices, tgt_indices):
    return _sc_lookup(
        src_table, tgt_table,
        src_indices.astype(jnp.int32), tgt_indices.astype(jnp.int32))


# speedup vs baseline: 5.9670x; 1.0019x over previous
import functools

import jax
import jax.numpy as jnp
from jax import lax
from jax.experimental import pallas as pl
from jax.experimental.pallas import tpu as pltpu
from jax.experimental.pallas import tpu_sc as plsc

_K = 8


def _sc_lookup(src_table, tgt_table, src_idx, tgt_idx):
    V, D = src_table.shape
    B, L = src_idx.shape
    info = plsc.get_sparse_core_info()
    NC = info.num_cores
    NW = NC * info.num_subcores
    per_w = B // NW
    n_outer = per_w // _K

    mesh = plsc.VectorSubcoreMesh(core_axis_name="c", subcore_axis_name="s")

    @functools.partial(
        pl.kernel,
        mesh=mesh,
        out_type=[
            jax.ShapeDtypeStruct((B, L, D), jnp.float32),
            jax.ShapeDtypeStruct((B, L, D), jnp.float32),
        ],
        scratch_types=[
            pltpu.VMEM((per_w, L), jnp.int32),
            pltpu.VMEM((per_w, L), jnp.int32),
            pltpu.VMEM((_K, L, D), jnp.float32),
            pltpu.SemaphoreType.DMA((_K,)),
            pltpu.SemaphoreType.DMA((_K,)),
        ],
    )
    def k(src_t, tgt_t, src_i, tgt_i, src_o, tgt_o,
          idx_src_v, idx_tgt_v, rows_v, sem_g, sem_w):
        wid = lax.axis_index("s") * NC + lax.axis_index("c")
        base = wid * per_w

        pltpu.sync_copy(src_i.at[pl.ds(base, per_w)], idx_src_v)
        pltpu.sync_copy(tgt_i.at[pl.ds(base, per_w)], idx_tgt_v)

        def side(table, idx_v, out_hbm, prev_out):
            def outer(t, carry):
                gathers = []
                for b in range(_K):
                    @pl.when(t > 0)
                    def _():
                        pltpu.make_async_copy(
                            rows_v.at[b], out_hbm.at[base], sem_w.at[b]
                        ).wait()

                    if prev_out is not None:
                        @pl.when(t == 0)
                        def _():
                            pltpu.make_async_copy(
                                rows_v.at[b], prev_out.at[base], sem_w.at[b]
                            ).wait()

                    c = t * _K + b
                    gathers.append(pltpu.async_copy(
                        table.at[idx_v.at[c]], rows_v.at[b], sem_g.at[b]))

                for b in range(_K):
                    gathers[b].wait()
                    c = t * _K + b
                    pltpu.async_copy(
                        rows_v.at[b], out_hbm.at[base + c], sem_w.at[b])
                return carry

            lax.fori_loop(0, n_outer, outer, 0)

        side(src_t, idx_src_v, src_o, None)
        side(tgt_t, idx_tgt_v, tgt_o, src_o)

        for b in range(_K):
            pltpu.make_async_copy(
                rows_v.at[b], tgt_o.at[base], sem_w.at[b]
            ).wait()

    return k(src_table, tgt_table, src_idx, tgt_idx)


def kernel(src_table, tgt_table, src_indices, tgt_indices):
    src_out, tgt_out = _sc_lookup(
        src_table, tgt_table,
        src_indices.astype(jnp.int32), tgt_indices.astype(jnp.int32))
    return (src_out, tgt_out)
